# baseline (speedup 1.0000x reference)
import functools

import jax
import jax.numpy as jnp
from jax import lax
from jax.experimental import pallas as pl
from jax.experimental.pallas import tpu as pltpu
from jax.experimental.pallas import tpu_sc as plsc

_N = 8192
_D = 2048
_NC = 2
_NS = 16
_NW = _NC * _NS
_RPW = _N // _NW
_BR = 16
_UNROLL = 8


def _sc_body(q_hbm, in_hbm, tgt_hbm, out_hbm, q_v, i_v, t_v, o_v):
    wid = lax.axis_index("s") * _NC + lax.axis_index("c")
    base = wid * _RPW
    pltpu.sync_copy(q_hbm.at[pl.ds(base, _RPW)], q_v)

    def outer(it, carry):
        r0 = base + it * _BR
        pltpu.sync_copy(in_hbm.at[pl.ds(r0, _BR)], i_v)
        pltpu.sync_copy(tgt_hbm.at[pl.ds(r0, _BR)], t_v)
        qv16 = q_v[pl.ds(it * _BR, 16)]
        for j in range(_BR):
            qs = qv16[j]

            def q1_chunks(j=j):
                def body(c, _):
                    for u in range(_UNROLL):
                        off = (c * _UNROLL + u) * 16
                        iv = i_v[j, pl.ds(off, 16)]
                        tv = t_v[j, pl.ds(off, 16)]
                        gap = tv * (iv - tv)
                        o_v[j, pl.ds(off, 16)] = gap * gap
                    return 0
                lax.fori_loop(0, _D // 16 // _UNROLL, body, 0)

            def q2_chunks(j=j):
                def body(c, _):
                    for u in range(_UNROLL):
                        off = (c * _UNROLL + u) * 16
                        gap = i_v[j, pl.ds(off, 16)] - t_v[j, pl.ds(off, 16)]
                        o_v[j, pl.ds(off, 16)] = gap * gap
                    return 0
                lax.fori_loop(0, _D // 16 // _UNROLL, body, 0)

            lax.cond(qs == 1, q1_chunks, q2_chunks)
        pltpu.sync_copy(o_v, out_hbm.at[pl.ds(r0, _BR)])
        return carry

    lax.fori_loop(0, _RPW // _BR, outer, 0)


def kernel(input_y, target_y, q):
    n, d = input_y.shape
    qi = q.astype(jnp.int32)
    mesh = plsc.VectorSubcoreMesh(core_axis_name="c", subcore_axis_name="s")
    sc = functools.partial(
        pl.kernel,
        out_type=jax.ShapeDtypeStruct((n, d), jnp.float32),
        mesh=mesh,
        scratch_types=[
            pltpu.VMEM((_RPW,), jnp.int32),
            pltpu.VMEM((_BR, d), jnp.float32),
            pltpu.VMEM((_BR, d), jnp.float32),
            pltpu.VMEM((_BR, d), jnp.float32),
        ],
    )(_sc_body)
    return sc(qi, input_y, target_y)


# speedup vs baseline: 1.7131x; 1.7131x over previous
import functools

import jax
import jax.numpy as jnp
from jax import lax
from jax.experimental import pallas as pl
from jax.experimental.pallas import tpu as pltpu
from jax.experimental.pallas import tpu_sc as plsc

_N = 8192
_D = 2048
_SC_ROWS = 2048
_NC = 2
_NS = 16
_NW = _NC * _NS
_RPW = _SC_ROWS // _NW
_BR = 16
_UNROLL = 8

_TC_BLOCK = 512


def _sc_body(q_hbm, in_hbm, tgt_hbm, out_hbm, q_v, i_v, t_v, o_v):
    wid = lax.axis_index("s") * _NC + lax.axis_index("c")
    base = wid * _RPW
    pltpu.sync_copy(q_hbm.at[pl.ds(base, _RPW)], q_v)

    def outer(it, carry):
        r0 = base + it * _BR
        pltpu.sync_copy(in_hbm.at[pl.ds(r0, _BR)], i_v)
        pltpu.sync_copy(tgt_hbm.at[pl.ds(r0, _BR)], t_v)
        qv16 = q_v[pl.ds(it * _BR, 16)]
        for j in range(_BR):
            qs = qv16[j]

            def q1_chunks(j=j):
                def body(c, _):
                    for u in range(_UNROLL):
                        off = (c * _UNROLL + u) * 16
                        iv = i_v[j, pl.ds(off, 16)]
                        tv = t_v[j, pl.ds(off, 16)]
                        gap = tv * (iv - tv)
                        o_v[j, pl.ds(off, 16)] = gap * gap
                    return 0
                lax.fori_loop(0, _D // 16 // _UNROLL, body, 0)

            def q2_chunks(j=j):
                def body(c, _):
                    for u in range(_UNROLL):
                        off = (c * _UNROLL + u) * 16
                        gap = i_v[j, pl.ds(off, 16)] - t_v[j, pl.ds(off, 16)]
                        o_v[j, pl.ds(off, 16)] = gap * gap
                    return 0
                lax.fori_loop(0, _D // 16 // _UNROLL, body, 0)

            lax.cond(qs == 1, q1_chunks, q2_chunks)
        pltpu.sync_copy(o_v, out_hbm.at[pl.ds(r0, _BR)])
        return carry

    lax.fori_loop(0, _RPW // _BR, outer, 0)


def _sc_part(qi, input_y, target_y):
    mesh = plsc.VectorSubcoreMesh(core_axis_name="c", subcore_axis_name="s")
    sc = functools.partial(
        pl.kernel,
        out_type=jax.ShapeDtypeStruct((_SC_ROWS, _D), jnp.float32),
        mesh=mesh,
        scratch_types=[
            pltpu.VMEM((_RPW,), jnp.int32),
            pltpu.VMEM((_BR, _D), jnp.float32),
            pltpu.VMEM((_BR, _D), jnp.float32),
            pltpu.VMEM((_BR, _D), jnp.float32),
        ],
    )(_sc_body)
    return sc(qi, input_y, target_y)


def _tc_gap_kernel(q_ref, in_ref, tgt_ref, out_ref):
    qv = q_ref[...]
    i = in_ref[...]
    t = tgt_ref[...]
    diff = i - t
    gap = jnp.where(qv == 1, t * diff, diff)
    out_ref[...] = gap * gap


def _tc_part(qi, input_y, target_y):
    b = _TC_BLOCK
    g = (_N - _SC_ROWS) // b
    o = _SC_ROWS // b
    m = qi.reshape(_N, 1)
    return pl.pallas_call(
        _tc_gap_kernel,
        grid=(g,),
        in_specs=[
            pl.BlockSpec((b, 1), lambda i: (i + o, 0)),
            pl.BlockSpec((b, _D), lambda i: (i + o, 0)),
            pl.BlockSpec((b, _D), lambda i: (i + o, 0)),
        ],
        out_specs=pl.BlockSpec((b, _D), lambda i: (i + o, 0)),
        out_shape=jax.ShapeDtypeStruct((_N, _D), jnp.float32),
        compiler_params=pltpu.CompilerParams(
            dimension_semantics=("arbitrary",),
        ),
    )(m, input_y, target_y)


def _copy_kernel(full_ref, sc_ref, out_ref):
    del full_ref
    out_ref[...] = sc_ref[...]


def _stitch(tc_out, sc_out):
    b = _TC_BLOCK
    g = _SC_ROWS // b
    return pl.pallas_call(
        _copy_kernel,
        grid=(g,),
        in_specs=[
            pl.BlockSpec(memory_space=pl.ANY),
            pl.BlockSpec((b, _D), lambda i: (i, 0)),
        ],
        out_specs=pl.BlockSpec((b, _D), lambda i: (i, 0)),
        out_shape=jax.ShapeDtypeStruct((_N, _D), jnp.float32),
        input_output_aliases={0: 0},
        compiler_params=pltpu.CompilerParams(
            dimension_semantics=("arbitrary",),
        ),
    )(tc_out, sc_out)


def kernel(input_y, target_y, q):
    qi = q.astype(jnp.int32)
    sc_out = _sc_part(qi, input_y, target_y)
    tc_out = _tc_part(qi, input_y, target_y)
    return _stitch(tc_out, sc_out)


# speedup vs baseline: 2.6311x; 1.5359x over previous
import jax
import jax.numpy as jnp
from jax.experimental import pallas as pl
from jax.experimental.pallas import tpu as pltpu

_BLOCK_ROWS = 512


def _gap_sq_kernel(q_ref, in_ref, tgt_ref, out_ref):
    qcol = q_ref[...].reshape(_BLOCK_ROWS, 1)
    i = in_ref[...]
    t = tgt_ref[...]
    diff = i - t
    gap = jnp.where(qcol == 1, t * diff, diff)
    out_ref[...] = gap * gap


def kernel(input_y, target_y, q):
    n, d = input_y.shape
    b = _BLOCK_ROWS
    g = n // b
    return pl.pallas_call(
        _gap_sq_kernel,
        grid=(g,),
        in_specs=[
            pl.BlockSpec((b,), lambda i: (i,)),
            pl.BlockSpec((b, d), lambda i: (i, 0)),
            pl.BlockSpec((b, d), lambda i: (i, 0)),
        ],
        out_specs=pl.BlockSpec((b, d), lambda i: (i, 0)),
        out_shape=jax.ShapeDtypeStruct((n, d), jnp.float32),
        compiler_params=pltpu.CompilerParams(
            dimension_semantics=("arbitrary",),
        ),
    )(q.astype(jnp.int32), input_y, target_y)
